# Initial kernel scaffold; baseline (speedup 1.0000x reference)
import jax
import jax.numpy as jnp
from jax.experimental import pallas as pl


def kernel(x, edge_index, params1, params2, params3):
    raise NotImplementedError("write your pallas kernel here")


# speedup vs baseline: 2.2049x; 2.2049x over previous
import jax
import jax.numpy as jnp
from jax.experimental import pallas as pl

_N = 10000
_ROW_BLK = 400


def _prop_matmul_kernel(s_ref, z_ref, o_ref):
    o_ref[...] = jnp.dot(s_ref[...], z_ref[...],
                         precision=jax.lax.Precision.HIGHEST,
                         preferred_element_type=jnp.float32)


def _prop(S, z):
    n, c = z.shape
    return pl.pallas_call(
        _prop_matmul_kernel,
        grid=(n // _ROW_BLK,),
        in_specs=[
            pl.BlockSpec((_ROW_BLK, n), lambda i: (i, 0)),
            pl.BlockSpec((n, c), lambda i: (0, 0)),
        ],
        out_specs=pl.BlockSpec((_ROW_BLK, c), lambda i: (i, 0)),
        out_shape=jax.ShapeDtypeStruct((n, c), jnp.float32),
    )(S, z)


def _dense_s(row, col, n):
    w = jnp.where(row != col, 1.0, 0.0).astype(jnp.float32)
    deg = jnp.zeros((n,), jnp.float32).at[row].add(w)
    dis = jnp.where(deg > 0, 1.0 / jnp.sqrt(jnp.where(deg > 0, deg, 1.0)), 0.0)
    norm = -dis[row] * w * dis[col]
    S = jnp.zeros((n, n), jnp.float32).at[col, row].add(norm)
    return S


def _cheb_conv(x, S, W, b):
    Tx0 = x
    out = Tx0 @ W[0]
    Tx1 = _prop(S, Tx0)
    out = out + Tx1 @ W[1]
    for k in range(2, W.shape[0]):
        Tx2 = 2.0 * _prop(S, Tx1) - Tx0
        out = out + Tx2 @ W[k]
        Tx0, Tx1 = Tx1, Tx2
    return out + b


def _branch(x, S, params):
    h = x
    for i, (W, b) in enumerate(params):
        h = _cheb_conv(h, S, W, b)
        if i < len(params) - 1:
            h = jax.nn.relu(h)
    return h


def kernel(x, edge_index, params1, params2, params3):
    n1 = _N
    n3 = edge_index.shape[1] // 3
    e1 = edge_index[:, 0:n3]
    e2 = edge_index[:, n3:2 * n3]
    e3 = edge_index[:, 2 * n3:]
    x1 = x[0:2 * n1:2, :]
    x2 = x[1:2 * n1:2, :]
    x3 = x[2 * n1:, :]
    S1 = _dense_s(e1[0], e1[1], n1)
    S2 = _dense_s(e2[0], e2[1], n1)
    S3 = _dense_s(e3[0], e3[1], x3.shape[0])
    o1 = _branch(x1, S1, params1)
    o2 = _branch(x2, S2, params2)
    o3 = _branch(x3, S3, params3)
    uv = jnp.stack([o1, o2], axis=1).reshape(2 * n1, o1.shape[1])
    return jnp.concatenate([uv, o3], axis=0)


# speedup vs baseline: 3.4923x; 1.5839x over previous
import jax
import jax.numpy as jnp
from jax.experimental import pallas as pl

_N = 10000
_ROW_BLK = 400


def _prop_matmul_kernel(shi_ref, slo_ref, z_ref, o_ref):
    z = z_ref[...]
    z_hi = z.astype(jnp.bfloat16)
    z_lo = (z - z_hi.astype(jnp.float32)).astype(jnp.bfloat16)
    s_hi = shi_ref[...]
    s_lo = slo_ref[...]
    acc = jnp.dot(s_hi, z_hi, preferred_element_type=jnp.float32)
    acc += jnp.dot(s_hi, z_lo, preferred_element_type=jnp.float32)
    acc += jnp.dot(s_lo, z_hi, preferred_element_type=jnp.float32)
    o_ref[...] = acc


def _prop(S_hi, S_lo, z):
    c = z.shape[1]
    n = S_hi.shape[0]
    return pl.pallas_call(
        _prop_matmul_kernel,
        grid=(n // _ROW_BLK,),
        in_specs=[
            pl.BlockSpec((_ROW_BLK, n), lambda i: (i, 0)),
            pl.BlockSpec((_ROW_BLK, n), lambda i: (i, 0)),
            pl.BlockSpec((n, c), lambda i: (0, 0)),
        ],
        out_specs=pl.BlockSpec((_ROW_BLK, c), lambda i: (i, 0)),
        out_shape=jax.ShapeDtypeStruct((n, c), jnp.float32),
    )(S_hi, S_lo, z)


def _dense_s(row, col, n):
    w = jnp.where(row != col, 1.0, 0.0).astype(jnp.float32)
    deg = jnp.zeros((n,), jnp.float32).at[row].add(w)
    dis = jnp.where(deg > 0, 1.0 / jnp.sqrt(jnp.where(deg > 0, deg, 1.0)), 0.0)
    norm = -dis[row] * w * dis[col]
    S = jnp.zeros((n, n), jnp.float32).at[col, row].add(norm)
    S_hi = S.astype(jnp.bfloat16)
    S_lo = (S - S_hi.astype(jnp.float32)).astype(jnp.bfloat16)
    return S_hi, S_lo


def _cheb_conv(x, S, W, b):
    S_hi, S_lo = S
    Tx0 = x
    out = Tx0 @ W[0]
    Tx1 = _prop(S_hi, S_lo, Tx0)
    out = out + Tx1 @ W[1]
    for k in range(2, W.shape[0]):
        Tx2 = 2.0 * _prop(S_hi, S_lo, Tx1) - Tx0
        out = out + Tx2 @ W[k]
        Tx0, Tx1 = Tx1, Tx2
    return out + b


def _branch(x, S, params):
    h = x
    for i, (W, b) in enumerate(params):
        h = _cheb_conv(h, S, W, b)
        if i < len(params) - 1:
            h = jax.nn.relu(h)
    return h


def kernel(x, edge_index, params1, params2, params3):
    n1 = _N
    n3 = edge_index.shape[1] // 3
    e1 = edge_index[:, 0:n3]
    e2 = edge_index[:, n3:2 * n3]
    e3 = edge_index[:, 2 * n3:]
    x1 = x[0:2 * n1:2, :]
    x2 = x[1:2 * n1:2, :]
    x3 = x[2 * n1:, :]
    S1 = _dense_s(e1[0], e1[1], n1)
    S2 = _dense_s(e2[0], e2[1], n1)
    S3 = _dense_s(e3[0], e3[1], x3.shape[0])
    o1 = _branch(x1, S1, params1)
    o2 = _branch(x2, S2, params2)
    o3 = _branch(x3, S3, params3)
    uv = jnp.stack([o1, o2], axis=1).reshape(2 * n1, o1.shape[1])
    return jnp.concatenate([uv, o3], axis=0)
